# baseline (speedup 1.0000x reference)
import functools

import jax
import jax.numpy as jnp
from jax import lax
from jax.experimental import pallas as pl
from jax.experimental.pallas import tpu as pltpu
from jax.experimental.pallas import tpu_sc as plsc

NC = 2
NS = 16
NW = NC * NS
CB = 100


def _mesh():
    return plsc.VectorSubcoreMesh(core_axis_name="c", subcore_axis_name="s")


def _sc_params():
    return pltpu.CompilerParams(use_tc_tiling_on_sc=False)


def _gather_body(table_h, idx_h, out_h, idx_v, big, sem, *, rw,
                 pass_sizes):
    c = lax.axis_index("c")
    s = lax.axis_index("s")
    wid = c * NS + s
    base = wid * rw
    pltpu.sync_copy(idx_h.at[wid], idx_v)
    off = 0
    for sz in pass_sizes:
        o = off

        def fire(j, carry, o=o):
            pltpu.async_copy(table_h.at[idx_v.at[o + j]],
                             big.at[pl.ds(j * CB, CB)], sem)
            return carry

        lax.fori_loop(0, sz, fire, 0)
        pltpu.make_async_copy(table_h.at[pl.ds(0, sz * CB)],
                              big.at[pl.ds(0, sz * CB)], sem).wait()
        pltpu.sync_copy(big.at[pl.ds(0, sz * CB)],
                        out_h.at[pl.ds((base + o) * CB, sz * CB)])
        off += sz


def _sc_gather(table, idx3d, npp):
    nw, rw, cb = idx3d.shape
    nn, d = table.shape
    e = nw * rw * cb
    pass_sizes = [npp] * (rw // npp)
    if rw % npp:
        pass_sizes.append(rw % npp)
    f = pl.kernel(
        functools.partial(_gather_body, rw=rw,
                          pass_sizes=tuple(pass_sizes)),
        out_type=jax.ShapeDtypeStruct((e, d), jnp.float32),
        mesh=_mesh(),
        scratch_types=[
            pltpu.VMEM((rw, cb), jnp.int32),
            pltpu.VMEM((npp * cb, d), jnp.float32),
            pltpu.SemaphoreType.DMA,
        ],
        compiler_params=_sc_params(),
    )
    return f(table, idx3d)


def _scatter_all_edges(msg_h, idx_h, idx_v, msg_v, sum_sh, cnt_sh, ones_v,
                       sem, csem, s, rw):
    for r in range(2):
        w2 = s * 2 + r
        pltpu.sync_copy(idx_h.at[w2], idx_v)
        pltpu.sync_copy(msg_h.at[pl.ds(w2 * rw * CB, rw * CB)], msg_v)

        def fire(j, carry):
            pltpu.async_copy(msg_v.at[pl.ds(j * CB, CB)],
                             sum_sh.at[idx_v.at[j]], sem, add=True)
            if cnt_sh is not None:
                pltpu.async_copy(ones_v, cnt_sh.at[idx_v.at[j]], csem,
                                 add=True)
            return carry

        lax.fori_loop(0, rw, fire, 0)
        pltpu.make_async_copy(msg_h.at[pl.ds(0, rw * CB)], msg_v, sem).wait()
        if cnt_sh is not None:
            pltpu.make_async_copy(msg_h.at[pl.ds(0, rw * CB)], msg_v,
                                  csem).wait()


def _layer0_sc_body(msg_h, dst_h, src4_h, zeros_h, ones_h, root_h,
                    x1two_h, inv_h, xj1_h,
                    idx_v, msg_v, ones_v,
                    sum_sh, cnt_sh, sem, csem, *, rw, n_nodes):
    c = lax.axis_index("c")
    s = lax.axis_index("s")
    rows = n_nodes // NS

    @pl.when(s == 0)
    def _():
        pltpu.sync_copy(zeros_h, sum_sh)
        pltpu.sync_copy(zeros_h, cnt_sh)

    pltpu.sync_copy(ones_h, ones_v)
    plsc.subcore_barrier()
    _scatter_all_edges(msg_h, dst_h, idx_v, msg_v, sum_sh, cnt_sh, ones_v,
                       sem, csem, s, rw)
    plsc.subcore_barrier()

    off = s * rows
    pltpu.sync_copy(sum_sh.at[pl.ds(off, rows)], msg_v.at[pl.ds(0, rows)])
    pltpu.sync_copy(cnt_sh.at[pl.ds(off, rows)],
                    msg_v.at[pl.ds(rows, rows)])
    pltpu.sync_copy(root_h.at[pl.ds(off, rows)],
                    msg_v.at[pl.ds(2 * rows, rows)])

    def epi(i, carry):
        sv = msg_v[i, :]
        cv = msg_v[rows + i, :]
        rv = msg_v[2 * rows + i, :]
        inv = 1.0 / jnp.maximum(cv, 1.0)
        msg_v[i, :] = jnp.maximum(sv * inv + rv, 0.0)
        msg_v[rows + i, :] = inv
        return carry

    lax.fori_loop(0, rows, epi, 0)
    pltpu.sync_copy(msg_v.at[pl.ds(0, rows)],
                    x1two_h.at[pl.ds(c * n_nodes + off, rows)])

    @pl.when(c == 0)
    def _():
        pltpu.sync_copy(msg_v.at[pl.ds(rows, rows)],
                        inv_h.at[pl.ds(off, rows)])

    plsc.subcore_barrier()

    wid = c * NS + s
    pltpu.sync_copy(src4_h.at[c, wid], idx_v)

    def gfire(j, carry):
        pltpu.async_copy(x1two_h.at[idx_v.at[j]],
                         msg_v.at[pl.ds(j * CB, CB)], sem)
        return carry

    lax.fori_loop(0, rw, gfire, 0)
    pltpu.make_async_copy(msg_h.at[pl.ds(0, rw * CB)], msg_v, sem).wait()
    pltpu.sync_copy(msg_v, xj1_h.at[pl.ds(wid * rw * CB, rw * CB)])


def _layer1_sc_body(msg_h, dst_h, zeros_h, inv_h, aggr_h,
                    idx_v, msg_v, sum_v, inv_v,
                    sum_sh, sem, *, rw, n_nodes):
    c = lax.axis_index("c")
    s = lax.axis_index("s")

    @pl.when(s == 0)
    def _():
        pltpu.sync_copy(zeros_h, sum_sh)

    plsc.subcore_barrier()
    _scatter_all_edges(msg_h, dst_h, idx_v, msg_v, sum_sh, None, None,
                       sem, None, s, rw)
    plsc.subcore_barrier()

    wid = c * NS + s
    per = 320
    off = jnp.minimum(wid * per, n_nodes - per)
    pltpu.sync_copy(sum_sh.at[pl.ds(off, per)], sum_v)
    pltpu.sync_copy(inv_h.at[pl.ds(off, per)], inv_v)

    def epi(i, carry):
        sum_v[i, :] = sum_v[i, :] * inv_v[i, :]
        return carry

    lax.fori_loop(0, per, epi, 0)
    pltpu.sync_copy(sum_v, aggr_h.at[pl.ds(off, per)])


def _sc_layer0(msg, dst3d, src4d, zeros, ones, root0, n_nodes):
    nw, rw, cb = dst3d.shape
    e = nw * rw * cb
    f = pl.kernel(
        functools.partial(_layer0_sc_body, rw=rw, n_nodes=n_nodes),
        out_type=[jax.ShapeDtypeStruct((NC * n_nodes, 16), jnp.float32),
                  jax.ShapeDtypeStruct((n_nodes, 16), jnp.float32),
                  jax.ShapeDtypeStruct((e, 16), jnp.float32)],
        mesh=_mesh(),
        scratch_types=[
            pltpu.VMEM((rw, cb), jnp.int32),
            pltpu.VMEM((rw * cb, 16), jnp.float32),
            pltpu.VMEM((cb, 16), jnp.float32),
            pltpu.VMEM_SHARED((n_nodes, 16), jnp.float32),
            pltpu.VMEM_SHARED((n_nodes, 16), jnp.float32),
            pltpu.SemaphoreType.DMA,
            pltpu.SemaphoreType.DMA,
        ],
        compiler_params=_sc_params(),
    )
    return f(msg, dst3d, src4d, zeros, ones, root0)


def _sc_layer1(msg, dst3d, zeros, inv, n_nodes):
    nw, rw, cb = dst3d.shape
    f = pl.kernel(
        functools.partial(_layer1_sc_body, rw=rw, n_nodes=n_nodes),
        out_type=jax.ShapeDtypeStruct((n_nodes, 16), jnp.float32),
        mesh=_mesh(),
        scratch_types=[
            pltpu.VMEM((rw, cb), jnp.int32),
            pltpu.VMEM((rw * cb, 16), jnp.float32),
            pltpu.VMEM((320, 16), jnp.float32),
            pltpu.VMEM((320, 16), jnp.float32),
            pltpu.VMEM_SHARED((n_nodes, 16), jnp.float32),
            pltpu.SemaphoreType.DMA,
        ],
        compiler_params=_sc_params(),
    )
    return f(msg, dst3d, zeros, inv)


def _msg_kernel(ea_ref, xj_ref, ew1t_ref, eb1_ref, at_ref, bt_ref, out_ref,
                *, h_dim, out_dim):
    ea_t = ea_ref[...].T
    xj_t = xj_ref[...].T
    h_t = jnp.dot(ew1t_ref[...], ea_t,
                  preferred_element_type=jnp.float32) + eb1_ref[...]
    h_t = h_t * jax.nn.sigmoid(h_t)
    g_t = jnp.dot(at_ref[...], xj_t,
                  preferred_element_type=jnp.float32)
    acc = jnp.dot(bt_ref[...], xj_t,
                  preferred_element_type=jnp.float32)
    for k in range(h_dim):
        acc = acc + h_t[k:k + 1, :] * g_t[k * out_dim:(k + 1) * out_dim, :]
    out_ref[...] = acc.T


def _tc_msg(ea, xj, ew1, eb1, ew2, eb2, in_dim, h_dim, out_dim, te):
    e = ea.shape[0]
    ed = ea.shape[1]
    at = ew2.reshape(h_dim, in_dim, out_dim).transpose(0, 2, 1).reshape(
        h_dim * out_dim, in_dim)
    bt = eb2.reshape(in_dim, out_dim).T
    grid = e // te
    return pl.pallas_call(
        functools.partial(_msg_kernel, h_dim=h_dim, out_dim=out_dim),
        grid=(grid,),
        in_specs=[
            pl.BlockSpec((te, ed), lambda i: (i, 0)),
            pl.BlockSpec((te, in_dim), lambda i: (i, 0)),
            pl.BlockSpec((h_dim, ed), lambda i: (0, 0)),
            pl.BlockSpec((h_dim, 1), lambda i: (0, 0)),
            pl.BlockSpec((h_dim * out_dim, in_dim), lambda i: (0, 0)),
            pl.BlockSpec((out_dim, in_dim), lambda i: (0, 0)),
        ],
        out_specs=pl.BlockSpec((te, out_dim), lambda i: (i, 0)),
        out_shape=jax.ShapeDtypeStruct((e, out_dim), jnp.float32),
    )(ea, xj, ew1.T, eb1.reshape(h_dim, 1), at, bt)


def _root_kernel(x_ref, root_ref, bias_ref, out_ref):
    out_ref[...] = jnp.dot(x_ref[...], root_ref[...],
                           preferred_element_type=jnp.float32) + bias_ref[...]


def _tc_root(x, root, bias):
    n = x.shape[0]
    h = root.shape[1]
    return pl.pallas_call(
        _root_kernel,
        out_shape=jax.ShapeDtypeStruct((n, h), jnp.float32),
    )(x, root, bias.reshape(1, h))


def _final_kernel(aggr_ref, x1_ref, root_ref, bias_ref,
                  w1_ref, b1_ref, w2_ref, b2_ref, out_ref):
    x2 = jnp.maximum(
        aggr_ref[...] + jnp.dot(x1_ref[...], root_ref[...],
                                preferred_element_type=jnp.float32)
        + bias_ref[...], 0.0)
    hmid = jnp.dot(x2, w1_ref[...], preferred_element_type=jnp.float32) \
        + b1_ref[...]
    hmid = hmid * jax.nn.sigmoid(hmid)
    out_ref[...] = jax.nn.sigmoid(
        jnp.dot(hmid, w2_ref[...], preferred_element_type=jnp.float32)
        + b2_ref[...])


def kernel(x, edge_index, edge_attr,
           l0_e_w1, l0_e_b1, l0_e_w2, l0_e_b2, l0_root, l0_bias,
           l1_e_w1, l1_e_b1, l1_e_w2, l1_e_b2, l1_root, l1_bias,
           mlp_w1, mlp_b1, mlp_w2, mlp_b2):
    n, in_dim = x.shape
    e, ed = edge_attr.shape
    h_dim = l0_root.shape[1]
    out_dim = mlp_w2.shape[1]
    te = 1000

    rw = e // (NW * CB)
    src3d = edge_index[0].reshape(NW, rw, CB)
    dst3d = edge_index[1].reshape(NW, rw, CB)
    src4d = jnp.stack([src3d, src3d + n])
    zeros = jnp.zeros((n, 16), jnp.float32)
    ones = jnp.ones((CB, 16), jnp.float32)

    xj0 = _sc_gather(x, src3d, npp=8)
    msg0 = _tc_msg(edge_attr, xj0, l0_e_w1, l0_e_b1, l0_e_w2, l0_e_b2,
                   in_dim, h_dim, h_dim, te)
    root0 = _tc_root(x, l0_root, l0_bias)
    x1two, inv, xj1 = _sc_layer0(msg0, dst3d, src4d, zeros, ones, root0, n)
    x1 = x1two[:n]

    msg1 = _tc_msg(edge_attr, xj1, l1_e_w1, l1_e_b1, l1_e_w2, l1_e_b2,
                   h_dim, h_dim, h_dim, te)
    aggr1 = _sc_layer1(msg1, dst3d, zeros, inv, n)
    out = pl.pallas_call(
        _final_kernel,
        out_shape=jax.ShapeDtypeStruct((n, out_dim), jnp.float32),
    )(aggr1, x1, l1_root, l1_bias.reshape(1, h_dim),
      mlp_w1, mlp_b1.reshape(1, h_dim), mlp_w2, mlp_b2.reshape(1, out_dim))
    return out


# speedup vs baseline: 1.1112x; 1.1112x over previous
import functools

import jax
import jax.numpy as jnp
from jax import lax
from jax.experimental import pallas as pl
from jax.experimental.pallas import tpu as pltpu
from jax.experimental.pallas import tpu_sc as plsc

NC = 2
NS = 16
NW = NC * NS
CB = 100


def _mesh():
    return plsc.VectorSubcoreMesh(core_axis_name="c", subcore_axis_name="s")


def _sc_params():
    return pltpu.CompilerParams(use_tc_tiling_on_sc=False)


def _gather_body(table_h, idx_h, out_h, idx_v, big, sem, *, rw,
                 pass_sizes):
    c = lax.axis_index("c")
    s = lax.axis_index("s")
    wid = c * NS + s
    base = wid * rw
    pltpu.sync_copy(idx_h.at[wid], idx_v)
    off = 0
    for sz in pass_sizes:
        o = off

        def fire(j, carry, o=o):
            pltpu.async_copy(table_h.at[idx_v.at[o + j]],
                             big.at[pl.ds(j * CB, CB)], sem)
            return carry

        lax.fori_loop(0, sz, fire, 0)
        pltpu.make_async_copy(table_h.at[pl.ds(0, sz * CB)],
                              big.at[pl.ds(0, sz * CB)], sem).wait()
        pltpu.sync_copy(big.at[pl.ds(0, sz * CB)],
                        out_h.at[pl.ds((base + o) * CB, sz * CB)])
        off += sz


def _sc_gather(table, idx3d, npp):
    nw, rw, cb = idx3d.shape
    nn, d = table.shape
    e = nw * rw * cb
    pass_sizes = [npp] * (rw // npp)
    if rw % npp:
        pass_sizes.append(rw % npp)
    f = pl.kernel(
        functools.partial(_gather_body, rw=rw,
                          pass_sizes=tuple(pass_sizes)),
        out_type=jax.ShapeDtypeStruct((e, d), jnp.float32),
        mesh=_mesh(),
        scratch_types=[
            pltpu.VMEM((rw, cb), jnp.int32),
            pltpu.VMEM((npp * cb, d), jnp.float32),
            pltpu.SemaphoreType.DMA,
        ],
        compiler_params=_sc_params(),
    )
    return f(table, idx3d)


def _scatter_all_edges(msg_h, idx_h, idx_v, msg_v, sum_sh, cnt_sh, ones_v,
                       sem, csem, s, rw):
    for r in range(2):
        w2 = s * 2 + r
        pltpu.sync_copy(idx_h.at[w2], idx_v)
        pltpu.sync_copy(msg_h.at[pl.ds(w2 * rw * CB, rw * CB)], msg_v)

        def fire(j, carry):
            pltpu.async_copy(msg_v.at[pl.ds(j * CB, CB)],
                             sum_sh.at[idx_v.at[j]], sem, add=True)
            if cnt_sh is not None:
                pltpu.async_copy(ones_v, cnt_sh.at[idx_v.at[j]], csem,
                                 add=True)
            return carry

        lax.fori_loop(0, rw, fire, 0)
        pltpu.make_async_copy(msg_h.at[pl.ds(0, rw * CB)], msg_v, sem).wait()
        if cnt_sh is not None:
            pltpu.make_async_copy(msg_h.at[pl.ds(0, rw * CB)], msg_v,
                                  csem).wait()


def _layer0_sc_body(msg_h, dst_h, src4_h, zeros_h, ones_h, root_h,
                    x1two_h, inv_h, xj1_h,
                    idx_v, msg_v, ones_v,
                    sum_sh, cnt_sh, sem, csem, *, rw, n_nodes):
    c = lax.axis_index("c")
    s = lax.axis_index("s")
    rows = n_nodes // NS

    @pl.when(s == 0)
    def _():
        pltpu.sync_copy(zeros_h, sum_sh)
        pltpu.sync_copy(zeros_h, cnt_sh)

    pltpu.sync_copy(ones_h, ones_v)
    plsc.subcore_barrier()
    _scatter_all_edges(msg_h, dst_h, idx_v, msg_v, sum_sh, cnt_sh, ones_v,
                       sem, csem, s, rw)
    plsc.subcore_barrier()

    off = s * rows
    pltpu.sync_copy(sum_sh.at[pl.ds(off, rows)], msg_v.at[pl.ds(0, rows)])
    pltpu.sync_copy(cnt_sh.at[pl.ds(off, rows)],
                    msg_v.at[pl.ds(rows, rows)])
    pltpu.sync_copy(root_h.at[pl.ds(off, rows)],
                    msg_v.at[pl.ds(2 * rows, rows)])

    def epi(i, carry):
        sv = msg_v[i, :]
        cv = msg_v[rows + i, :]
        rv = msg_v[2 * rows + i, :]
        inv = 1.0 / jnp.maximum(cv, 1.0)
        msg_v[i, :] = jnp.maximum(sv * inv + rv, 0.0)
        msg_v[rows + i, :] = inv
        return carry

    lax.fori_loop(0, rows, epi, 0)
    pltpu.sync_copy(msg_v.at[pl.ds(0, rows)],
                    x1two_h.at[pl.ds(c * n_nodes + off, rows)])

    @pl.when(c == 0)
    def _():
        pltpu.sync_copy(msg_v.at[pl.ds(rows, rows)],
                        inv_h.at[pl.ds(off, rows)])

    plsc.subcore_barrier()

    wid = c * NS + s
    pltpu.sync_copy(src4_h.at[c, wid], idx_v)

    def gfire(j, carry):
        pltpu.async_copy(x1two_h.at[idx_v.at[j]],
                         msg_v.at[pl.ds(j * CB, CB)], sem)
        return carry

    lax.fori_loop(0, rw, gfire, 0)
    pltpu.make_async_copy(msg_h.at[pl.ds(0, rw * CB)], msg_v, sem).wait()
    pltpu.sync_copy(msg_v, xj1_h.at[pl.ds(wid * rw * CB, rw * CB)])


def _layer1_sc_body(msg_h, dst_h, zeros_h, inv_h, aggr_h,
                    idx_v, msg_v, sum_v, inv_v,
                    sum_sh, sem, *, rw, n_nodes):
    c = lax.axis_index("c")
    s = lax.axis_index("s")

    @pl.when(s == 0)
    def _():
        pltpu.sync_copy(zeros_h, sum_sh)

    plsc.subcore_barrier()
    _scatter_all_edges(msg_h, dst_h, idx_v, msg_v, sum_sh, None, None,
                       sem, None, s, rw)
    plsc.subcore_barrier()

    wid = c * NS + s
    per = 320
    off = jnp.minimum(wid * per, n_nodes - per)
    pltpu.sync_copy(sum_sh.at[pl.ds(off, per)], sum_v)
    pltpu.sync_copy(inv_h.at[pl.ds(off, per)], inv_v)

    def epi(i, carry):
        sum_v[i, :] = sum_v[i, :] * inv_v[i, :]
        return carry

    lax.fori_loop(0, per, epi, 0)
    pltpu.sync_copy(sum_v, aggr_h.at[pl.ds(off, per)])


def _sc_layer0(msg, dst3d, src4d, zeros, ones, root0, n_nodes):
    nw, rw, cb = dst3d.shape
    e = nw * rw * cb
    f = pl.kernel(
        functools.partial(_layer0_sc_body, rw=rw, n_nodes=n_nodes),
        out_type=[jax.ShapeDtypeStruct((NC * n_nodes, 16), jnp.float32),
                  jax.ShapeDtypeStruct((n_nodes, 16), jnp.float32),
                  jax.ShapeDtypeStruct((e, 16), jnp.float32)],
        mesh=_mesh(),
        scratch_types=[
            pltpu.VMEM((rw, cb), jnp.int32),
            pltpu.VMEM((rw * cb, 16), jnp.float32),
            pltpu.VMEM((cb, 16), jnp.float32),
            pltpu.VMEM_SHARED((n_nodes, 16), jnp.float32),
            pltpu.VMEM_SHARED((n_nodes, 16), jnp.float32),
            pltpu.SemaphoreType.DMA,
            pltpu.SemaphoreType.DMA,
        ],
        compiler_params=_sc_params(),
    )
    return f(msg, dst3d, src4d, zeros, ones, root0)


def _sc_layer1(msg, dst3d, zeros, inv, n_nodes):
    nw, rw, cb = dst3d.shape
    f = pl.kernel(
        functools.partial(_layer1_sc_body, rw=rw, n_nodes=n_nodes),
        out_type=jax.ShapeDtypeStruct((n_nodes, 16), jnp.float32),
        mesh=_mesh(),
        scratch_types=[
            pltpu.VMEM((rw, cb), jnp.int32),
            pltpu.VMEM((rw * cb, 16), jnp.float32),
            pltpu.VMEM((320, 16), jnp.float32),
            pltpu.VMEM((320, 16), jnp.float32),
            pltpu.VMEM_SHARED((n_nodes, 16), jnp.float32),
            pltpu.SemaphoreType.DMA,
        ],
        compiler_params=_sc_params(),
    )
    return f(msg, dst3d, zeros, inv)


def _msg_kernel(eat_ref, xjt_ref, ew1t_ref, eb1_ref, at_ref, bt_ref, out_ref,
                *, h_dim, out_dim):
    h_t = jnp.dot(ew1t_ref[...], eat_ref[...],
                  preferred_element_type=jnp.float32) + eb1_ref[...]
    h_t = h_t * jax.nn.sigmoid(h_t)
    g_t = jnp.dot(at_ref[...], xjt_ref[...],
                  preferred_element_type=jnp.float32)
    acc = jnp.dot(bt_ref[...], xjt_ref[...],
                  preferred_element_type=jnp.float32)
    for k in range(h_dim):
        acc = acc + h_t[k:k + 1, :] * g_t[k * out_dim:(k + 1) * out_dim, :]
    out_ref[...] = acc


def _tc_msg(ea_t, xj_t, ew1, eb1, ew2, eb2, in_dim, h_dim, out_dim, te):
    ed, e = ea_t.shape
    at = ew2.reshape(h_dim, in_dim, out_dim).transpose(0, 2, 1).reshape(
        h_dim * out_dim, in_dim)
    bt = eb2.reshape(in_dim, out_dim).T
    grid = e // te
    return pl.pallas_call(
        functools.partial(_msg_kernel, h_dim=h_dim, out_dim=out_dim),
        grid=(grid,),
        in_specs=[
            pl.BlockSpec((ed, te), lambda i: (0, i)),
            pl.BlockSpec((in_dim, te), lambda i: (0, i)),
            pl.BlockSpec((h_dim, ed), lambda i: (0, 0)),
            pl.BlockSpec((h_dim, 1), lambda i: (0, 0)),
            pl.BlockSpec((h_dim * out_dim, in_dim), lambda i: (0, 0)),
            pl.BlockSpec((out_dim, in_dim), lambda i: (0, 0)),
        ],
        out_specs=pl.BlockSpec((out_dim, te), lambda i: (0, i)),
        out_shape=jax.ShapeDtypeStruct((out_dim, e), jnp.float32),
    )(ea_t, xj_t, ew1.T, eb1.reshape(h_dim, 1), at, bt)


def _root_kernel(x_ref, root_ref, bias_ref, out_ref):
    out_ref[...] = jnp.dot(x_ref[...], root_ref[...],
                           preferred_element_type=jnp.float32) + bias_ref[...]


def _tc_root(x, root, bias):
    n = x.shape[0]
    h = root.shape[1]
    return pl.pallas_call(
        _root_kernel,
        out_shape=jax.ShapeDtypeStruct((n, h), jnp.float32),
    )(x, root, bias.reshape(1, h))


def _final_kernel(aggr_ref, x1_ref, root_ref, bias_ref,
                  w1_ref, b1_ref, w2_ref, b2_ref, out_ref):
    x2 = jnp.maximum(
        aggr_ref[...] + jnp.dot(x1_ref[...], root_ref[...],
                                preferred_element_type=jnp.float32)
        + bias_ref[...], 0.0)
    hmid = jnp.dot(x2, w1_ref[...], preferred_element_type=jnp.float32) \
        + b1_ref[...]
    hmid = hmid * jax.nn.sigmoid(hmid)
    out_ref[...] = jax.nn.sigmoid(
        jnp.dot(hmid, w2_ref[...], preferred_element_type=jnp.float32)
        + b2_ref[...])


def kernel(x, edge_index, edge_attr,
           l0_e_w1, l0_e_b1, l0_e_w2, l0_e_b2, l0_root, l0_bias,
           l1_e_w1, l1_e_b1, l1_e_w2, l1_e_b2, l1_root, l1_bias,
           mlp_w1, mlp_b1, mlp_w2, mlp_b2):
    n, in_dim = x.shape
    e, ed = edge_attr.shape
    h_dim = l0_root.shape[1]
    out_dim = mlp_w2.shape[1]
    te = 1280

    rw = e // (NW * CB)
    src3d = edge_index[0].reshape(NW, rw, CB)
    dst3d = edge_index[1].reshape(NW, rw, CB)
    src4d = jnp.stack([src3d, src3d + n])
    zeros = jnp.zeros((n, 16), jnp.float32)
    ones = jnp.ones((CB, 16), jnp.float32)

    ea_t = edge_attr.T

    xj0 = _sc_gather(x, src3d, npp=8)
    msg0_t = _tc_msg(ea_t, xj0.T, l0_e_w1, l0_e_b1, l0_e_w2, l0_e_b2,
                     in_dim, h_dim, h_dim, te)
    root0 = _tc_root(x, l0_root, l0_bias)
    x1two, inv, xj1 = _sc_layer0(msg0_t.T, dst3d, src4d, zeros, ones,
                                 root0, n)
    x1 = x1two[:n]

    msg1_t = _tc_msg(ea_t, xj1.T, l1_e_w1, l1_e_b1, l1_e_w2, l1_e_b2,
                     h_dim, h_dim, h_dim, te)
    aggr1 = _sc_layer1(msg1_t.T, dst3d, zeros, inv, n)
    out = pl.pallas_call(
        _final_kernel,
        out_shape=jax.ShapeDtypeStruct((n, out_dim), jnp.float32),
    )(aggr1, x1, l1_root, l1_bias.reshape(1, h_dim),
      mlp_w1, mlp_b1.reshape(1, h_dim), mlp_w2, mlp_b2.reshape(1, out_dim))
    return out


# speedup vs baseline: 1.2595x; 1.1334x over previous
import functools

import jax
import jax.numpy as jnp
from jax import lax
from jax.experimental import pallas as pl
from jax.experimental.pallas import tpu as pltpu
from jax.experimental.pallas import tpu_sc as plsc

NC = 2
NS = 16
NW = NC * NS
CB = 100


def _mesh():
    return plsc.VectorSubcoreMesh(core_axis_name="c", subcore_axis_name="s")


def _sc_params():
    return pltpu.CompilerParams(use_tc_tiling_on_sc=False)


def _gather_body(table_h, idx_h, out_h, idx_v, big, sem, *, rw,
                 pass_sizes):
    c = lax.axis_index("c")
    s = lax.axis_index("s")
    wid = c * NS + s
    base = wid * rw
    pltpu.sync_copy(idx_h.at[wid], idx_v)
    off = 0
    for sz in pass_sizes:
        o = off

        def fire(j, carry, o=o):
            pltpu.async_copy(table_h.at[idx_v.at[o + j]],
                             big.at[pl.ds(j * CB, CB)], sem)
            return carry

        lax.fori_loop(0, sz, fire, 0)
        pltpu.make_async_copy(table_h.at[pl.ds(0, sz * CB)],
                              big.at[pl.ds(0, sz * CB)], sem).wait()
        pltpu.sync_copy(big.at[pl.ds(0, sz * CB)],
                        out_h.at[pl.ds((base + o) * CB, sz * CB)])
        off += sz


def _sc_gather(table, idx3d, npp):
    nw, rw, cb = idx3d.shape
    nn, d = table.shape
    e = nw * rw * cb
    pass_sizes = [npp] * (rw // npp)
    if rw % npp:
        pass_sizes.append(rw % npp)
    f = pl.kernel(
        functools.partial(_gather_body, rw=rw,
                          pass_sizes=tuple(pass_sizes)),
        out_type=jax.ShapeDtypeStruct((e, d), jnp.float32),
        mesh=_mesh(),
        scratch_types=[
            pltpu.VMEM((rw, cb), jnp.int32),
            pltpu.VMEM((npp * cb, d), jnp.float32),
            pltpu.SemaphoreType.DMA,
        ],
        compiler_params=_sc_params(),
    )
    return f(table, idx3d)


def _scatter_all_edges(msg_h, idx_h, idx_v, msg_v, sum_sh, cnt_sh, ones_v,
                       sem, csem, s, rw):
    for r in range(2):
        w2 = s * 2 + r
        pltpu.sync_copy(idx_h.at[w2], idx_v)
        pltpu.sync_copy(msg_h.at[pl.ds(w2 * rw * CB, rw * CB)], msg_v)

        def fire(j, carry):
            pltpu.async_copy(msg_v.at[pl.ds(j * CB, CB)],
                             sum_sh.at[idx_v.at[j]], sem, add=True)
            if cnt_sh is not None:
                pltpu.async_copy(ones_v, cnt_sh.at[idx_v.at[j]], csem,
                                 add=True)
            return carry

        lax.fori_loop(0, rw, fire, 0)
        pltpu.make_async_copy(msg_h.at[pl.ds(0, rw * CB)], msg_v, sem).wait()
        if cnt_sh is not None:
            pltpu.make_async_copy(msg_h.at[pl.ds(0, rw * CB)], msg_v,
                                  csem).wait()


def _layer0_sc_body(msg_h, dst_h, src4_h, zeros_h, ones_h, root_h,
                    x1two_h, inv_h, xj1_h,
                    idx_v, msg_v, ones_v,
                    sum_sh, cnt_sh, sem, csem, *, rw, n_nodes):
    c = lax.axis_index("c")
    s = lax.axis_index("s")
    rows = n_nodes // NS

    @pl.when(s == 0)
    def _():
        pltpu.sync_copy(zeros_h, sum_sh)
        pltpu.sync_copy(zeros_h, cnt_sh)

    pltpu.sync_copy(ones_h, ones_v)
    plsc.subcore_barrier()
    _scatter_all_edges(msg_h, dst_h, idx_v, msg_v, sum_sh, cnt_sh, ones_v,
                       sem, csem, s, rw)
    plsc.subcore_barrier()

    off = s * rows
    pltpu.sync_copy(sum_sh.at[pl.ds(off, rows)], msg_v.at[pl.ds(0, rows)])
    pltpu.sync_copy(cnt_sh.at[pl.ds(off, rows)],
                    msg_v.at[pl.ds(rows, rows)])
    pltpu.sync_copy(root_h.at[pl.ds(off, rows)],
                    msg_v.at[pl.ds(2 * rows, rows)])

    def epi(i, carry):
        sv = msg_v[i, :]
        cv = msg_v[rows + i, :]
        rv = msg_v[2 * rows + i, :]
        inv = 1.0 / jnp.maximum(cv, 1.0)
        msg_v[i, :] = jnp.maximum(sv * inv + rv, 0.0)
        msg_v[rows + i, :] = inv
        return carry

    lax.fori_loop(0, rows, epi, 0)
    pltpu.sync_copy(msg_v.at[pl.ds(0, rows)],
                    x1two_h.at[pl.ds(c * n_nodes + off, rows)])

    @pl.when(c == 0)
    def _():
        pltpu.sync_copy(msg_v.at[pl.ds(rows, rows)],
                        inv_h.at[pl.ds(off, rows)])

    plsc.subcore_barrier()

    wid = c * NS + s
    pltpu.sync_copy(src4_h.at[c, wid], idx_v)

    def gfire(j, carry):
        pltpu.async_copy(x1two_h.at[idx_v.at[j]],
                         msg_v.at[pl.ds(j * CB, CB)], sem)
        return carry

    lax.fori_loop(0, rw, gfire, 0)
    pltpu.make_async_copy(msg_h.at[pl.ds(0, rw * CB)], msg_v, sem).wait()
    pltpu.sync_copy(msg_v, xj1_h.at[pl.ds(wid * rw * CB, rw * CB)])


def _layer1_sc_body(msg_h, dst_h, zeros_h, inv_h, aggr_h,
                    idx_v, msg_v, sum_v, inv_v,
                    sum_sh, sem, *, rw, n_nodes):
    c = lax.axis_index("c")
    s = lax.axis_index("s")

    @pl.when(s == 0)
    def _():
        pltpu.sync_copy(zeros_h, sum_sh)

    plsc.subcore_barrier()
    _scatter_all_edges(msg_h, dst_h, idx_v, msg_v, sum_sh, None, None,
                       sem, None, s, rw)
    plsc.subcore_barrier()

    wid = c * NS + s
    per = 320
    off = jnp.minimum(wid * per, n_nodes - per)
    pltpu.sync_copy(sum_sh.at[pl.ds(off, per)], sum_v)
    pltpu.sync_copy(inv_h.at[pl.ds(off, per)], inv_v)

    def epi(i, carry):
        sum_v[i, :] = sum_v[i, :] * inv_v[i, :]
        return carry

    lax.fori_loop(0, per, epi, 0)
    pltpu.sync_copy(sum_v, aggr_h.at[pl.ds(off, per)])


def _sc_layer0(msg, dst3d, src4d, zeros, ones, root0, n_nodes):
    nw, rw, cb = dst3d.shape
    e = nw * rw * cb
    f = pl.kernel(
        functools.partial(_layer0_sc_body, rw=rw, n_nodes=n_nodes),
        out_type=[jax.ShapeDtypeStruct((NC * n_nodes, 16), jnp.float32),
                  jax.ShapeDtypeStruct((n_nodes, 16), jnp.float32),
                  jax.ShapeDtypeStruct((e, 16), jnp.float32)],
        mesh=_mesh(),
        scratch_types=[
            pltpu.VMEM((rw, cb), jnp.int32),
            pltpu.VMEM((rw * cb, 16), jnp.float32),
            pltpu.VMEM((cb, 16), jnp.float32),
            pltpu.VMEM_SHARED((n_nodes, 16), jnp.float32),
            pltpu.VMEM_SHARED((n_nodes, 16), jnp.float32),
            pltpu.SemaphoreType.DMA,
            pltpu.SemaphoreType.DMA,
        ],
        compiler_params=_sc_params(),
    )
    return f(msg, dst3d, src4d, zeros, ones, root0)


def _sc_layer1(msg, dst3d, zeros, inv, n_nodes):
    nw, rw, cb = dst3d.shape
    f = pl.kernel(
        functools.partial(_layer1_sc_body, rw=rw, n_nodes=n_nodes),
        out_type=jax.ShapeDtypeStruct((n_nodes, 16), jnp.float32),
        mesh=_mesh(),
        scratch_types=[
            pltpu.VMEM((rw, cb), jnp.int32),
            pltpu.VMEM((rw * cb, 16), jnp.float32),
            pltpu.VMEM((320, 16), jnp.float32),
            pltpu.VMEM((320, 16), jnp.float32),
            pltpu.VMEM_SHARED((n_nodes, 16), jnp.float32),
            pltpu.SemaphoreType.DMA,
        ],
        compiler_params=_sc_params(),
    )
    return f(msg, dst3d, zeros, inv)


def _msg_kernel(eat_ref, xjt_ref, ew1t_ref, eb1_ref, at_ref, bt_ref, out_ref,
                *, h_dim, out_dim):
    h_t = jnp.dot(ew1t_ref[...], eat_ref[...],
                  preferred_element_type=jnp.float32) + eb1_ref[...]
    h_t = h_t * jax.nn.sigmoid(h_t)
    g_t = jnp.dot(at_ref[...], xjt_ref[...],
                  preferred_element_type=jnp.float32)
    acc = jnp.dot(bt_ref[...], xjt_ref[...],
                  preferred_element_type=jnp.float32)
    for k in range(h_dim):
        acc = acc + h_t[k:k + 1, :] * g_t[k * out_dim:(k + 1) * out_dim, :]
    out_ref[...] = acc


def _tc_msg(ea_t, xj_t, ew1, eb1, ew2, eb2, in_dim, h_dim, out_dim, te):
    ed, e = ea_t.shape
    at = ew2.reshape(h_dim, in_dim, out_dim).transpose(0, 2, 1).reshape(
        h_dim * out_dim, in_dim)
    bt = eb2.reshape(in_dim, out_dim).T
    grid = e // te
    return pl.pallas_call(
        functools.partial(_msg_kernel, h_dim=h_dim, out_dim=out_dim),
        grid=(grid,),
        in_specs=[
            pl.BlockSpec((ed, te), lambda i: (0, i)),
            pl.BlockSpec((in_dim, te), lambda i: (0, i)),
            pl.BlockSpec((h_dim, ed), lambda i: (0, 0)),
            pl.BlockSpec((h_dim, 1), lambda i: (0, 0)),
            pl.BlockSpec((h_dim * out_dim, in_dim), lambda i: (0, 0)),
            pl.BlockSpec((out_dim, in_dim), lambda i: (0, 0)),
        ],
        out_specs=pl.BlockSpec((out_dim, te), lambda i: (0, i)),
        out_shape=jax.ShapeDtypeStruct((out_dim, e), jnp.float32),
    )(ea_t, xj_t, ew1.T, eb1.reshape(h_dim, 1), at, bt)


def _root_kernel(x_ref, root_ref, bias_ref, out_ref):
    out_ref[...] = jnp.dot(x_ref[...], root_ref[...],
                           preferred_element_type=jnp.float32) + bias_ref[...]


def _tc_root(x, root, bias):
    n = x.shape[0]
    h = root.shape[1]
    return pl.pallas_call(
        _root_kernel,
        out_shape=jax.ShapeDtypeStruct((n, h), jnp.float32),
    )(x, root, bias.reshape(1, h))


def _final_kernel(aggr_ref, x1_ref, root_ref, bias_ref,
                  w1_ref, b1_ref, w2_ref, b2_ref, out_ref):
    x2 = jnp.maximum(
        aggr_ref[...] + jnp.dot(x1_ref[...], root_ref[...],
                                preferred_element_type=jnp.float32)
        + bias_ref[...], 0.0)
    hmid = jnp.dot(x2, w1_ref[...], preferred_element_type=jnp.float32) \
        + b1_ref[...]
    hmid = hmid * jax.nn.sigmoid(hmid)
    out_ref[...] = jax.nn.sigmoid(
        jnp.dot(hmid, w2_ref[...], preferred_element_type=jnp.float32)
        + b2_ref[...])


def kernel(x, edge_index, edge_attr,
           l0_e_w1, l0_e_b1, l0_e_w2, l0_e_b2, l0_root, l0_bias,
           l1_e_w1, l1_e_b1, l1_e_w2, l1_e_b2, l1_root, l1_bias,
           mlp_w1, mlp_b1, mlp_w2, mlp_b2):
    n, in_dim = x.shape
    e, ed = edge_attr.shape
    h_dim = l0_root.shape[1]
    out_dim = mlp_w2.shape[1]
    te = 3200

    rw = e // (NW * CB)
    src3d = edge_index[0].reshape(NW, rw, CB)
    dst3d = edge_index[1].reshape(NW, rw, CB)
    src4d = jnp.stack([src3d, src3d + n])
    zeros = jnp.zeros((n, 16), jnp.float32)
    ones = jnp.ones((CB, 16), jnp.float32)

    ea_t = edge_attr.T

    xj0 = _sc_gather(x, src3d, npp=10)
    msg0_t = _tc_msg(ea_t, xj0.T, l0_e_w1, l0_e_b1, l0_e_w2, l0_e_b2,
                     in_dim, h_dim, h_dim, te)
    root0 = _tc_root(x, l0_root, l0_bias)
    x1two, inv, xj1 = _sc_layer0(msg0_t.T, dst3d, src4d, zeros, ones,
                                 root0, n)
    x1 = x1two[:n]

    msg1_t = _tc_msg(ea_t, xj1.T, l1_e_w1, l1_e_b1, l1_e_w2, l1_e_b2,
                     h_dim, h_dim, h_dim, te)
    aggr1 = _sc_layer1(msg1_t.T, dst3d, zeros, inv, n)
    out = pl.pallas_call(
        _final_kernel,
        out_shape=jax.ShapeDtypeStruct((n, out_dim), jnp.float32),
    )(aggr1, x1, l1_root, l1_bias.reshape(1, h_dim),
      mlp_w1, mlp_b1.reshape(1, h_dim), mlp_w2, mlp_b2.reshape(1, out_dim))
    return out


# speedup vs baseline: 1.3144x; 1.0436x over previous
import functools

import jax
import jax.numpy as jnp
from jax import lax
from jax.experimental import pallas as pl
from jax.experimental.pallas import tpu as pltpu
from jax.experimental.pallas import tpu_sc as plsc

NC = 2
NS = 16
NW = NC * NS
CB = 100


def _mesh():
    return plsc.VectorSubcoreMesh(core_axis_name="c", subcore_axis_name="s")


def _sc_params():
    return pltpu.CompilerParams(use_tc_tiling_on_sc=False)


def _gather_body(table_h, idx_h, out_h, idx_v, big, sem, *, rw,
                 pass_sizes):
    c = lax.axis_index("c")
    s = lax.axis_index("s")
    wid = c * NS + s
    base = wid * rw
    pltpu.sync_copy(idx_h.at[wid], idx_v)
    off = 0
    for sz in pass_sizes:
        o = off

        def fire(j, carry, o=o):
            pltpu.async_copy(table_h.at[idx_v.at[o + j]],
                             big.at[pl.ds(j * CB, CB)], sem)
            return carry

        lax.fori_loop(0, sz, fire, 0)
        pltpu.make_async_copy(table_h.at[pl.ds(0, sz * CB)],
                              big.at[pl.ds(0, sz * CB)], sem).wait()
        pltpu.sync_copy(big.at[pl.ds(0, sz * CB)],
                        out_h.at[pl.ds((base + o) * CB, sz * CB)])
        off += sz


def _sc_gather(table, idx3d, npp):
    nw, rw, cb = idx3d.shape
    nn, d = table.shape
    e = nw * rw * cb
    pass_sizes = [npp] * (rw // npp)
    if rw % npp:
        pass_sizes.append(rw % npp)
    f = pl.kernel(
        functools.partial(_gather_body, rw=rw,
                          pass_sizes=tuple(pass_sizes)),
        out_type=jax.ShapeDtypeStruct((e, d), jnp.float32),
        mesh=_mesh(),
        scratch_types=[
            pltpu.VMEM((rw, cb), jnp.int32),
            pltpu.VMEM((npp * cb, d), jnp.float32),
            pltpu.SemaphoreType.DMA,
        ],
        compiler_params=_sc_params(),
    )
    return f(table, idx3d)


def _scatter_all_edges(msg_h, idx_h, idx_v, msg_v, sum_sh, cnt_sh, ones_v,
                       sem, csem, s, rw):
    for r in range(2):
        w2 = s * 2 + r
        pltpu.sync_copy(idx_h.at[w2], idx_v)
        pltpu.sync_copy(msg_h.at[pl.ds(w2 * rw * CB, rw * CB)], msg_v)

        def fire(j, carry):
            pltpu.async_copy(msg_v.at[pl.ds(j * CB, CB)],
                             sum_sh.at[idx_v.at[j]], sem, add=True)
            if cnt_sh is not None:
                pltpu.async_copy(ones_v, cnt_sh.at[idx_v.at[j]], csem,
                                 add=True)
            return carry

        lax.fori_loop(0, rw, fire, 0)
        pltpu.make_async_copy(msg_h.at[pl.ds(0, rw * CB)], msg_v, sem).wait()
        if cnt_sh is not None:
            pltpu.make_async_copy(msg_h.at[pl.ds(0, rw * CB)], msg_v,
                                  csem).wait()


def _layer0_sc_body(msg_h, dst_h, src4_h, zeros_h, ones_h, root_h,
                    x1two_h, inv_h, xj1_h,
                    idx_v, msg_v, ones_v,
                    sum_sh, cnt_sh, sem, csem, *, rw, n_nodes):
    c = lax.axis_index("c")
    s = lax.axis_index("s")
    rows = n_nodes // NS

    @pl.when(s == 0)
    def _():
        pltpu.sync_copy(zeros_h, sum_sh)
        pltpu.sync_copy(zeros_h, cnt_sh)

    pltpu.sync_copy(ones_h, ones_v)
    plsc.subcore_barrier()
    _scatter_all_edges(msg_h, dst_h, idx_v, msg_v, sum_sh, cnt_sh, ones_v,
                       sem, csem, s, rw)
    plsc.subcore_barrier()

    off = s * rows
    pltpu.sync_copy(sum_sh.at[pl.ds(off, rows)], msg_v.at[pl.ds(0, rows)])
    pltpu.sync_copy(cnt_sh.at[pl.ds(off, rows)],
                    msg_v.at[pl.ds(rows, rows)])
    pltpu.sync_copy(root_h.at[pl.ds(off, rows)],
                    msg_v.at[pl.ds(2 * rows, rows)])

    def epi(i, carry):
        sv = msg_v[i, :]
        cv = msg_v[rows + i, :]
        rv = msg_v[2 * rows + i, :]
        inv = 1.0 / jnp.maximum(cv, 1.0)
        msg_v[i, :] = jnp.maximum(sv * inv + rv, 0.0)
        msg_v[rows + i, :] = inv
        return carry

    lax.fori_loop(0, rows, epi, 0)
    pltpu.sync_copy(msg_v.at[pl.ds(0, rows)],
                    x1two_h.at[pl.ds(c * n_nodes + off, rows)])

    @pl.when(c == 0)
    def _():
        pltpu.sync_copy(msg_v.at[pl.ds(rows, rows)],
                        inv_h.at[pl.ds(off, rows)])

    plsc.subcore_barrier()

    wid = c * NS + s
    pltpu.sync_copy(src4_h.at[c, wid], idx_v)

    def gfire(j, carry):
        pltpu.async_copy(x1two_h.at[idx_v.at[j]],
                         msg_v.at[pl.ds(j * CB, CB)], sem)
        return carry

    lax.fori_loop(0, rw, gfire, 0)
    pltpu.make_async_copy(msg_h.at[pl.ds(0, rw * CB)], msg_v, sem).wait()
    pltpu.sync_copy(msg_v, xj1_h.at[pl.ds(wid * rw * CB, rw * CB)])


def _layer1_sc_body(msg_h, dst_h, zeros_h, inv_h, aggr_h,
                    idx_v, msg_v, sum_v, inv_v,
                    sum_sh, sem, *, rw, n_nodes):
    c = lax.axis_index("c")
    s = lax.axis_index("s")

    @pl.when(s == 0)
    def _():
        pltpu.sync_copy(zeros_h, sum_sh)

    plsc.subcore_barrier()
    _scatter_all_edges(msg_h, dst_h, idx_v, msg_v, sum_sh, None, None,
                       sem, None, s, rw)
    plsc.subcore_barrier()

    wid = c * NS + s
    per = 320
    off = jnp.minimum(wid * per, n_nodes - per)
    pltpu.sync_copy(sum_sh.at[pl.ds(off, per)], sum_v)
    pltpu.sync_copy(inv_h.at[pl.ds(off, per)], inv_v)

    def epi(i, carry):
        sum_v[i, :] = sum_v[i, :] * inv_v[i, :]
        return carry

    lax.fori_loop(0, per, epi, 0)
    pltpu.sync_copy(sum_v, aggr_h.at[pl.ds(off, per)])


def _sc_layer0(msg, dst3d, src4d, zeros, ones, root0, n_nodes):
    nw, rw, cb = dst3d.shape
    e = nw * rw * cb
    f = pl.kernel(
        functools.partial(_layer0_sc_body, rw=rw, n_nodes=n_nodes),
        out_type=[jax.ShapeDtypeStruct((NC * n_nodes, 16), jnp.float32),
                  jax.ShapeDtypeStruct((n_nodes, 16), jnp.float32),
                  jax.ShapeDtypeStruct((e, 16), jnp.float32)],
        mesh=_mesh(),
        scratch_types=[
            pltpu.VMEM((rw, cb), jnp.int32),
            pltpu.VMEM((rw * cb, 16), jnp.float32),
            pltpu.VMEM((cb, 16), jnp.float32),
            pltpu.VMEM_SHARED((n_nodes, 16), jnp.float32),
            pltpu.VMEM_SHARED((n_nodes, 16), jnp.float32),
            pltpu.SemaphoreType.DMA,
            pltpu.SemaphoreType.DMA,
        ],
        compiler_params=_sc_params(),
    )
    return f(msg, dst3d, src4d, zeros, ones, root0)


def _sc_layer1(msg, dst3d, zeros, inv, n_nodes):
    nw, rw, cb = dst3d.shape
    f = pl.kernel(
        functools.partial(_layer1_sc_body, rw=rw, n_nodes=n_nodes),
        out_type=jax.ShapeDtypeStruct((n_nodes, 16), jnp.float32),
        mesh=_mesh(),
        scratch_types=[
            pltpu.VMEM((rw, cb), jnp.int32),
            pltpu.VMEM((rw * cb, 16), jnp.float32),
            pltpu.VMEM((320, 16), jnp.float32),
            pltpu.VMEM((320, 16), jnp.float32),
            pltpu.VMEM_SHARED((n_nodes, 16), jnp.float32),
            pltpu.SemaphoreType.DMA,
        ],
        compiler_params=_sc_params(),
    )
    return f(msg, dst3d, zeros, inv)


def _msg_kernel(eat_ref, xjt_ref, ew1t_ref, eb1_ref, at_ref, bt_ref, out_ref,
                *, h_dim, out_dim):
    h_t = jnp.dot(ew1t_ref[...], eat_ref[...],
                  preferred_element_type=jnp.float32) + eb1_ref[...]
    h_t = h_t * jax.nn.sigmoid(h_t)
    g_t = jnp.dot(at_ref[...], xjt_ref[...],
                  preferred_element_type=jnp.float32)
    acc = jnp.dot(bt_ref[...], xjt_ref[...],
                  preferred_element_type=jnp.float32)
    for k in range(h_dim):
        acc = acc + h_t[k:k + 1, :] * g_t[k * out_dim:(k + 1) * out_dim, :]
    out_ref[...] = acc


def _tc_msg(ea_t, xj_t, ew1, eb1, ew2, eb2, in_dim, h_dim, out_dim, te):
    ed, e = ea_t.shape
    at = ew2.reshape(h_dim, in_dim, out_dim).transpose(0, 2, 1).reshape(
        h_dim * out_dim, in_dim)
    bt = eb2.reshape(in_dim, out_dim).T
    grid = e // te
    return pl.pallas_call(
        functools.partial(_msg_kernel, h_dim=h_dim, out_dim=out_dim),
        grid=(grid,),
        in_specs=[
            pl.BlockSpec((ed, te), lambda i: (0, i)),
            pl.BlockSpec((in_dim, te), lambda i: (0, i)),
            pl.BlockSpec((h_dim, ed), lambda i: (0, 0)),
            pl.BlockSpec((h_dim, 1), lambda i: (0, 0)),
            pl.BlockSpec((h_dim * out_dim, in_dim), lambda i: (0, 0)),
            pl.BlockSpec((out_dim, in_dim), lambda i: (0, 0)),
        ],
        out_specs=pl.BlockSpec((out_dim, te), lambda i: (0, i)),
        out_shape=jax.ShapeDtypeStruct((out_dim, e), jnp.float32),
    )(ea_t, xj_t, ew1.T, eb1.reshape(h_dim, 1), at, bt)


def _root_kernel(x_ref, root_ref, bias_ref, out_ref):
    out_ref[...] = jnp.dot(x_ref[...], root_ref[...],
                           preferred_element_type=jnp.float32) + bias_ref[...]


def _tc_root(x, root, bias):
    n = x.shape[0]
    h = root.shape[1]
    return pl.pallas_call(
        _root_kernel,
        out_shape=jax.ShapeDtypeStruct((n, h), jnp.float32),
    )(x, root, bias.reshape(1, h))


def _final_kernel(aggr_ref, x1_ref, root_ref, bias_ref,
                  w1_ref, b1_ref, w2_ref, b2_ref, out_ref):
    x2 = jnp.maximum(
        aggr_ref[...] + jnp.dot(x1_ref[...], root_ref[...],
                                preferred_element_type=jnp.float32)
        + bias_ref[...], 0.0)
    hmid = jnp.dot(x2, w1_ref[...], preferred_element_type=jnp.float32) \
        + b1_ref[...]
    hmid = hmid * jax.nn.sigmoid(hmid)
    out_ref[...] = jax.nn.sigmoid(
        jnp.dot(hmid, w2_ref[...], preferred_element_type=jnp.float32)
        + b2_ref[...])


def kernel(x, edge_index, edge_attr,
           l0_e_w1, l0_e_b1, l0_e_w2, l0_e_b2, l0_root, l0_bias,
           l1_e_w1, l1_e_b1, l1_e_w2, l1_e_b2, l1_root, l1_bias,
           mlp_w1, mlp_b1, mlp_w2, mlp_b2):
    n, in_dim = x.shape
    e, ed = edge_attr.shape
    h_dim = l0_root.shape[1]
    out_dim = mlp_w2.shape[1]
    te = 6400

    rw = e // (NW * CB)
    src3d = edge_index[0].reshape(NW, rw, CB)
    dst3d = edge_index[1].reshape(NW, rw, CB)
    src4d = jnp.stack([src3d, src3d + n])
    zeros = jnp.zeros((n, 16), jnp.float32)
    ones = jnp.ones((CB, 16), jnp.float32)

    ea_t = edge_attr.T

    xj0 = _sc_gather(x, src3d, npp=10)
    msg0_t = _tc_msg(ea_t, xj0.T, l0_e_w1, l0_e_b1, l0_e_w2, l0_e_b2,
                     in_dim, h_dim, h_dim, te)
    root0 = _tc_root(x, l0_root, l0_bias)
    x1two, inv, xj1 = _sc_layer0(msg0_t.T, dst3d, src4d, zeros, ones,
                                 root0, n)
    x1 = x1two[:n]

    msg1_t = _tc_msg(ea_t, xj1.T, l1_e_w1, l1_e_b1, l1_e_w2, l1_e_b2,
                     h_dim, h_dim, h_dim, te)
    aggr1 = _sc_layer1(msg1_t.T, dst3d, zeros, inv, n)
    out = pl.pallas_call(
        _final_kernel,
        out_shape=jax.ShapeDtypeStruct((n, out_dim), jnp.float32),
    )(aggr1, x1, l1_root, l1_bias.reshape(1, h_dim),
      mlp_w1, mlp_b1.reshape(1, h_dim), mlp_w2, mlp_b2.reshape(1, out_dim))
    return out
